# baseline (speedup 1.0000x reference)
import functools
import math

import jax
import jax.numpy as jnp
from jax import lax
from jax.experimental import pallas as pl
from jax.experimental.pallas import tpu as pltpu
from jax.experimental.pallas import tpu_sc as plsc

E = 8
TOPK = 2
D = 768
FF = 3072
CF = 1.25
T = 2048
CAP = int(math.ceil(CF * T * TOPK / E))
EB = CAP + 8
NROWS = E * EB

NC, NS = 2, 16
NW = NC * NS
TPW = T // NW
RBLK = 512
FBLK = 3072
NFB = FF // FBLK


def _route_body(xt_ref, wg_ref, bg_ref, d0_ref, d1_ref, wr0_ref, wr1_ref,
                oh_s, meta_s, tblk_s):
    pid = pl.program_id(0)
    nblk = T // RBLK

    @pl.when(pid == 0)
    def _():
        xt = xt_ref[...]
        wg = wg_ref[...]
        logits = lax.dot_general(xt, wg, (((1,), (1,)), ((), ())),
                                 preferred_element_type=jnp.float32)
        logits = logits + bg_ref[...]
        iota8 = lax.broadcasted_iota(jnp.int32, (T, E), 1).astype(jnp.float32)
        m1 = jnp.max(logits, axis=1, keepdims=True)
        a1 = jnp.min(jnp.where(logits == m1, iota8, 1e9), axis=1, keepdims=True)
        masked = jnp.where(iota8 == a1, -1e30, logits)
        m2 = jnp.max(masked, axis=1, keepdims=True)
        a2 = jnp.min(jnp.where(masked == m2, iota8, 1e9), axis=1, keepdims=True)
        p = jnp.exp(logits - m1)
        denom = jnp.sum(p, axis=1, keepdims=True)
        g0 = 1.0 / denom
        g1 = jnp.exp(m2 - m1) / denom
        oh0 = (iota8 == a1).astype(jnp.float32)
        oh1 = (iota8 == a2).astype(jnp.float32)
        oh = jnp.concatenate([oh0, oh1], axis=1)
        oh_s[...] = oh
        meta_s[...] = jnp.concatenate([a1, a2, g0, g1], axis=1)
        srow = lax.broadcasted_iota(jnp.int32, (8, T), 0)
        scol = lax.broadcasted_iota(jnp.int32, (8, T), 1) // RBLK
        sel = (srow == scol).astype(jnp.float32)
        tblk_s[...] = lax.dot_general(sel, oh, (((1,), (0,)), ((), ())),
                                      preferred_element_type=jnp.float32)

    row = lax.broadcasted_iota(jnp.int32, (RBLK, RBLK), 0)
    col = lax.broadcasted_iota(jnp.int32, (RBLK, RBLK), 1)
    a_tri = (col <= row).astype(jnp.float32)
    ohb = oh_s[pl.ds(pid * RBLK, RBLK), :]
    c_loc = lax.dot_general(a_tri, ohb, (((1,), (0,)), ((), ())),
                            preferred_element_type=jnp.float32)
    tblk = tblk_s[...]
    bidx = lax.broadcasted_iota(jnp.int32, (8, 2 * E), 0)
    prefix = jnp.sum(jnp.where(bidx < pid, tblk, 0.0), axis=0, keepdims=True)
    totals = jnp.sum(tblk, axis=0, keepdims=True)
    c_inc = c_loc + prefix
    r0 = jnp.sum(c_inc[:, :E] * ohb[:, :E], axis=1, keepdims=True) - 1.0
    r1 = jnp.sum(c_inc[:, E:] * ohb[:, E:], axis=1, keepdims=True) - 1.0
    count0 = jnp.minimum(totals[:, :E], float(CAP))
    cnt_at = jnp.sum(ohb[:, E:] * count0, axis=1, keepdims=True)
    pos1 = r1 + cnt_at
    kept0 = r0 < CAP
    kept1 = pos1 < CAP
    meta = meta_s[pl.ds(pid * RBLK, RBLK), :]
    a1 = meta[:, 0:1]
    a2 = meta[:, 1:2]
    g0 = meta[:, 2:3]
    g1 = meta[:, 3:4]
    d0 = a1 * EB + jnp.where(kept0, r0, float(CAP))
    d1 = a2 * EB + jnp.where(kept1, pos1, float(CAP))
    d0_ref[...] = lax.transpose(d0.astype(jnp.int32), (1, 0))
    d1_ref[...] = lax.transpose(d1.astype(jnp.int32), (1, 0))
    wr0_ref[...] = jnp.broadcast_to(jnp.where(kept0, g0, 0.0), (RBLK, 128))
    wr1_ref[...] = jnp.broadcast_to(jnp.where(kept1, g1, 0.0), (RBLK, 128))


def _route(xt, wg, bg):
    d0, d1, wr0, wr1 = pl.pallas_call(
        _route_body,
        grid=(T // RBLK,),
        in_specs=[
            pl.BlockSpec((T, D), lambda i: (0, 0)),
            pl.BlockSpec((E, D), lambda i: (0, 0)),
            pl.BlockSpec((1, E), lambda i: (0, 0)),
        ],
        out_specs=(
            pl.BlockSpec((1, RBLK), lambda i: (0, i)),
            pl.BlockSpec((1, RBLK), lambda i: (0, i)),
            pl.BlockSpec((RBLK, 128), lambda i: (i, 0)),
            pl.BlockSpec((RBLK, 128), lambda i: (i, 0)),
        ),
        out_shape=(
            jax.ShapeDtypeStruct((1, T), jnp.int32),
            jax.ShapeDtypeStruct((1, T), jnp.int32),
            jax.ShapeDtypeStruct((T, 128), jnp.float32),
            jax.ShapeDtypeStruct((T, 128), jnp.float32),
        ),
        scratch_shapes=[
            pltpu.VMEM((T, 2 * E), jnp.float32),
            pltpu.VMEM((T, 4), jnp.float32),
            pltpu.VMEM((8, 2 * E), jnp.float32),
        ],
    )(xt, wg, bg.reshape(1, E))
    return d0, d1, wr0, wr1


HLF = TPW // 2


def _sc_dispatch_body(xt_hbm, d0_hbm, d1_hbm, wr0_hbm, wr1_hbm,
                      ein_hbm, sw_hbm, xv, w0v, w1v, i0v, i1v,
                      sema, semb, semw, sem0, sem1, sem2, sem3, semsw):
    wid = lax.axis_index("s") * NC + lax.axis_index("c")
    base = wid * TPW
    sa = pltpu.async_copy(xt_hbm.at[pl.ds(base, HLF)], xv.at[pl.ds(0, HLF)],
                          sema)
    sb = pltpu.async_copy(xt_hbm.at[pl.ds(base + HLF, HLF)],
                          xv.at[pl.ds(HLF, HLF)], semb)
    pltpu.sync_copy(d0_hbm.at[pl.ds(base, HLF)], i0v.at[0])
    pltpu.sync_copy(d0_hbm.at[pl.ds(base + HLF, HLF)], i0v.at[1])
    pltpu.sync_copy(d1_hbm.at[pl.ds(base, HLF)], i1v.at[0])
    pltpu.sync_copy(d1_hbm.at[pl.ds(base + HLF, HLF)], i1v.at[1])
    pltpu.sync_copy(wr0_hbm.at[pl.ds(base, TPW)], w0v)
    pltpu.sync_copy(wr1_hbm.at[pl.ds(base, TPW)], w1v)
    s0 = pltpu.async_copy(w0v.at[pl.ds(0, HLF)], sw_hbm.at[i0v.at[0]], semw)
    s1 = pltpu.async_copy(w0v.at[pl.ds(HLF, HLF)], sw_hbm.at[i0v.at[1]], semw)
    s2 = pltpu.async_copy(w1v.at[pl.ds(0, HLF)], sw_hbm.at[i1v.at[0]], semsw)
    s3 = pltpu.async_copy(w1v.at[pl.ds(HLF, HLF)], sw_hbm.at[i1v.at[1]], semsw)
    sa.wait()
    c0 = pltpu.async_copy(xv.at[pl.ds(0, HLF)], ein_hbm.at[i0v.at[0]], sem0)
    c1 = pltpu.async_copy(xv.at[pl.ds(0, HLF)], ein_hbm.at[i1v.at[0]], sem1)
    sb.wait()
    c2 = pltpu.async_copy(xv.at[pl.ds(HLF, HLF)], ein_hbm.at[i0v.at[1]], sem2)
    c3 = pltpu.async_copy(xv.at[pl.ds(HLF, HLF)], ein_hbm.at[i1v.at[1]], sem3)
    s0.wait()
    s1.wait()
    s2.wait()
    s3.wait()
    c0.wait()
    c1.wait()
    c2.wait()
    c3.wait()


def _sc_dispatch(xt, d0, d1, wr0, wr1):
    mesh = plsc.VectorSubcoreMesh(core_axis_name="c", subcore_axis_name="s",
                                  num_cores=NC, num_subcores=NS)
    return pl.kernel(
        _sc_dispatch_body,
        out_type=(
            jax.ShapeDtypeStruct((NROWS, D), jnp.float32),
            jax.ShapeDtypeStruct((NROWS, 128), jnp.float32),
        ),
        mesh=mesh,
        scratch_types=[
            pltpu.VMEM((TPW, D), jnp.float32),
            pltpu.VMEM((TPW, 128), jnp.float32),
            pltpu.VMEM((TPW, 128), jnp.float32),
            pltpu.VMEM((2, HLF), jnp.int32),
            pltpu.VMEM((2, HLF), jnp.int32),
            pltpu.SemaphoreType.DMA,
            pltpu.SemaphoreType.DMA,
            pltpu.SemaphoreType.DMA,
            pltpu.SemaphoreType.DMA,
            pltpu.SemaphoreType.DMA,
            pltpu.SemaphoreType.DMA,
            pltpu.SemaphoreType.DMA,
            pltpu.SemaphoreType.DMA,
        ],
    )(xt, d0, d1, wr0, wr1)


def _ffn_body(a_ref, sw_ref, w1_ref, b1_ref, w2_ref, b2_ref, y_ref):
    a = a_ref[...].astype(jnp.bfloat16)
    w1 = w1_ref[0].astype(jnp.bfloat16)
    h = jnp.maximum(
        lax.dot_general(a, w1, (((1,), (1,)), ((), ())),
                        preferred_element_type=jnp.float32)
        + b1_ref[0], 0.0).astype(jnp.bfloat16)
    w2 = w2_ref[0].astype(jnp.bfloat16)
    y = lax.dot_general(h, w2, (((1,), (1,)), ((), ())),
                        preferred_element_type=jnp.float32) + b2_ref[0]
    y_ref[...] = y * sw_ref[:, 0:1]


def _ffn(ein, sw, w1, b1, w2, b2):
    return pl.pallas_call(
        _ffn_body,
        grid=(E,),
        in_specs=[
            pl.BlockSpec((EB, D), lambda e: (e, 0)),
            pl.BlockSpec((EB, 128), lambda e: (e, 0)),
            pl.BlockSpec((1, FF, D), lambda e: (e, 0, 0)),
            pl.BlockSpec((1, 1, FF), lambda e: (e, 0, 0)),
            pl.BlockSpec((1, D, FF), lambda e: (e, 0, 0)),
            pl.BlockSpec((1, 1, D), lambda e: (e, 0, 0)),
        ],
        out_specs=pl.BlockSpec((EB, D), lambda e: (e, 0)),
        out_shape=jax.ShapeDtypeStruct((NROWS, D), jnp.float32),
    )(ein, sw, w1, b1.reshape(E, 1, FF), w2, b2.reshape(E, 1, D))


def _sc_combine_body(y_hbm, d0_hbm, d1_hbm, out_hbm,
                     r0v, r1v, i0v, i1v,
                     sem0, sem1, sem2, sem3, semo):
    wid = lax.axis_index("s") * NC + lax.axis_index("c")
    base = wid * TPW
    pltpu.sync_copy(d0_hbm.at[pl.ds(base, HLF)], i0v.at[0])
    pltpu.sync_copy(d0_hbm.at[pl.ds(base + HLF, HLF)], i0v.at[1])
    pltpu.sync_copy(d1_hbm.at[pl.ds(base, HLF)], i1v.at[0])
    pltpu.sync_copy(d1_hbm.at[pl.ds(base + HLF, HLF)], i1v.at[1])
    c0 = pltpu.async_copy(y_hbm.at[i0v.at[0]], r0v.at[pl.ds(0, HLF)], sem0)
    c1 = pltpu.async_copy(y_hbm.at[i1v.at[0]], r1v.at[pl.ds(0, HLF)], sem1)
    c2 = pltpu.async_copy(y_hbm.at[i0v.at[1]], r0v.at[pl.ds(HLF, HLF)], sem2)
    c3 = pltpu.async_copy(y_hbm.at[i1v.at[1]], r1v.at[pl.ds(HLF, HLF)], sem3)

    def body(j, carry):
        for c in range(D // 16):
            sl = pl.ds(c * 16, 16)
            r0v[j, sl] = r0v[j, sl] + r1v[j, sl]
        return carry

    c0.wait()
    c1.wait()
    lax.fori_loop(0, HLF, body, 0)
    oa = pltpu.async_copy(r0v.at[pl.ds(0, HLF)],
                          out_hbm.at[pl.ds(base, HLF)], semo)
    c2.wait()
    c3.wait()
    lax.fori_loop(HLF, TPW, body, 0)
    oa.wait()
    pltpu.sync_copy(r0v.at[pl.ds(HLF, HLF)],
                    out_hbm.at[pl.ds(base + HLF, HLF)])


def _sc_combine(y, d0, d1):
    mesh = plsc.VectorSubcoreMesh(core_axis_name="c", subcore_axis_name="s",
                                  num_cores=NC, num_subcores=NS)
    return pl.kernel(
        _sc_combine_body,
        out_type=jax.ShapeDtypeStruct((T, D), jnp.float32),
        mesh=mesh,
        scratch_types=[
            pltpu.VMEM((TPW, D), jnp.float32),
            pltpu.VMEM((TPW, D), jnp.float32),
            pltpu.VMEM((2, HLF), jnp.int32),
            pltpu.VMEM((2, HLF), jnp.int32),
            pltpu.SemaphoreType.DMA,
            pltpu.SemaphoreType.DMA,
            pltpu.SemaphoreType.DMA,
            pltpu.SemaphoreType.DMA,
            pltpu.SemaphoreType.DMA,
        ],
    )(y, d0, d1)


def kernel(x, Wg, bg, W1, B1, W2, B2):
    xt = x.reshape(T, D)
    d0c, d1c, wr0, wr1 = _route(xt, Wg, bg)
    d0 = d0c.reshape(T)
    d1 = d1c.reshape(T)
    ein, sw = _sc_dispatch(xt, d0, d1, wr0, wr1)
    y = _ffn(ein, sw, W1, B1, W2, B2)
    out = _sc_combine(y, d0, d1)
    return out.reshape(1, T, D)


# speedup vs baseline: 1.0056x; 1.0056x over previous
import functools
import math

import jax
import jax.numpy as jnp
from jax import lax
from jax.experimental import pallas as pl
from jax.experimental.pallas import tpu as pltpu
from jax.experimental.pallas import tpu_sc as plsc

E = 8
TOPK = 2
D = 768
FF = 3072
CF = 1.25
T = 2048
CAP = int(math.ceil(CF * T * TOPK / E))
EB = CAP + 8
NROWS = E * EB

NC, NS = 2, 16
NW = NC * NS
TPW = T // NW
RBLK = 512
FBLK = 3072
NFB = FF // FBLK


def _route_body(xt_ref, wg_ref, bg_ref, d0_ref, d1_ref, wr0_ref, wr1_ref,
                oh_s, meta_s, tblk_s):
    pid = pl.program_id(0)
    nblk = T // RBLK

    @pl.when(pid == 0)
    def _():
        xt = xt_ref[...]
        wg = wg_ref[...]
        logits = lax.dot_general(xt, wg, (((1,), (1,)), ((), ())),
                                 preferred_element_type=jnp.float32)
        logits = logits + bg_ref[...]
        iota8 = lax.broadcasted_iota(jnp.int32, (T, E), 1).astype(jnp.float32)
        m1 = jnp.max(logits, axis=1, keepdims=True)
        a1 = jnp.min(jnp.where(logits == m1, iota8, 1e9), axis=1, keepdims=True)
        masked = jnp.where(iota8 == a1, -1e30, logits)
        m2 = jnp.max(masked, axis=1, keepdims=True)
        a2 = jnp.min(jnp.where(masked == m2, iota8, 1e9), axis=1, keepdims=True)
        p = jnp.exp(logits - m1)
        denom = jnp.sum(p, axis=1, keepdims=True)
        g0 = 1.0 / denom
        g1 = jnp.exp(m2 - m1) / denom
        oh0 = (iota8 == a1).astype(jnp.float32)
        oh1 = (iota8 == a2).astype(jnp.float32)
        oh = jnp.concatenate([oh0, oh1], axis=1)
        oh_s[...] = oh
        meta_s[...] = jnp.concatenate([a1, a2, g0, g1], axis=1)
        srow = lax.broadcasted_iota(jnp.int32, (8, T), 0)
        scol = lax.broadcasted_iota(jnp.int32, (8, T), 1) // RBLK
        sel = (srow == scol).astype(jnp.float32)
        tblk_s[...] = lax.dot_general(sel, oh, (((1,), (0,)), ((), ())),
                                      preferred_element_type=jnp.float32)

    row = lax.broadcasted_iota(jnp.int32, (RBLK, RBLK), 0)
    col = lax.broadcasted_iota(jnp.int32, (RBLK, RBLK), 1)
    a_tri = (col <= row).astype(jnp.float32)
    ohb = oh_s[pl.ds(pid * RBLK, RBLK), :]
    c_loc = lax.dot_general(a_tri, ohb, (((1,), (0,)), ((), ())),
                            preferred_element_type=jnp.float32)
    tblk = tblk_s[...]
    bidx = lax.broadcasted_iota(jnp.int32, (8, 2 * E), 0)
    prefix = jnp.sum(jnp.where(bidx < pid, tblk, 0.0), axis=0, keepdims=True)
    totals = jnp.sum(tblk, axis=0, keepdims=True)
    c_inc = c_loc + prefix
    r0 = jnp.sum(c_inc[:, :E] * ohb[:, :E], axis=1, keepdims=True) - 1.0
    r1 = jnp.sum(c_inc[:, E:] * ohb[:, E:], axis=1, keepdims=True) - 1.0
    count0 = jnp.minimum(totals[:, :E], float(CAP))
    cnt_at = jnp.sum(ohb[:, E:] * count0, axis=1, keepdims=True)
    pos1 = r1 + cnt_at
    kept0 = r0 < CAP
    kept1 = pos1 < CAP
    meta = meta_s[pl.ds(pid * RBLK, RBLK), :]
    a1 = meta[:, 0:1]
    a2 = meta[:, 1:2]
    g0 = meta[:, 2:3]
    g1 = meta[:, 3:4]
    d0 = a1 * EB + jnp.where(kept0, r0, float(CAP))
    d1 = a2 * EB + jnp.where(kept1, pos1, float(CAP))
    d0_ref[...] = lax.transpose(d0.astype(jnp.int32), (1, 0))
    d1_ref[...] = lax.transpose(d1.astype(jnp.int32), (1, 0))
    wr0_ref[...] = jnp.broadcast_to(jnp.where(kept0, g0, 0.0), (RBLK, 16))
    wr1_ref[...] = jnp.broadcast_to(jnp.where(kept1, g1, 0.0), (RBLK, 16))


def _route(xt, wg, bg):
    d0, d1, wr0, wr1 = pl.pallas_call(
        _route_body,
        grid=(T // RBLK,),
        in_specs=[
            pl.BlockSpec((T, D), lambda i: (0, 0)),
            pl.BlockSpec((E, D), lambda i: (0, 0)),
            pl.BlockSpec((1, E), lambda i: (0, 0)),
        ],
        out_specs=(
            pl.BlockSpec((1, RBLK), lambda i: (0, i)),
            pl.BlockSpec((1, RBLK), lambda i: (0, i)),
            pl.BlockSpec((RBLK, 16), lambda i: (i, 0)),
            pl.BlockSpec((RBLK, 16), lambda i: (i, 0)),
        ),
        out_shape=(
            jax.ShapeDtypeStruct((1, T), jnp.int32),
            jax.ShapeDtypeStruct((1, T), jnp.int32),
            jax.ShapeDtypeStruct((T, 16), jnp.float32),
            jax.ShapeDtypeStruct((T, 16), jnp.float32),
        ),
        scratch_shapes=[
            pltpu.VMEM((T, 2 * E), jnp.float32),
            pltpu.VMEM((T, 4), jnp.float32),
            pltpu.VMEM((8, 2 * E), jnp.float32),
        ],
    )(xt, wg, bg.reshape(1, E))
    return d0, d1, wr0, wr1


HLF = TPW // 2


def _sc_dispatch_body(xt_hbm, d0_hbm, d1_hbm, ein_hbm, xv, i0v, i1v,
                      sema, semb, sem0, sem1, sem2, sem3):
    wid = lax.axis_index("s") * NC + lax.axis_index("c")
    base = wid * TPW
    sa = pltpu.async_copy(xt_hbm.at[pl.ds(base, HLF)], xv.at[pl.ds(0, HLF)],
                          sema)
    sb = pltpu.async_copy(xt_hbm.at[pl.ds(base + HLF, HLF)],
                          xv.at[pl.ds(HLF, HLF)], semb)
    pltpu.sync_copy(d0_hbm.at[pl.ds(base, HLF)], i0v.at[0])
    pltpu.sync_copy(d0_hbm.at[pl.ds(base + HLF, HLF)], i0v.at[1])
    pltpu.sync_copy(d1_hbm.at[pl.ds(base, HLF)], i1v.at[0])
    pltpu.sync_copy(d1_hbm.at[pl.ds(base + HLF, HLF)], i1v.at[1])
    sa.wait()
    c0 = pltpu.async_copy(xv.at[pl.ds(0, HLF)], ein_hbm.at[i0v.at[0]], sem0)
    c1 = pltpu.async_copy(xv.at[pl.ds(0, HLF)], ein_hbm.at[i1v.at[0]], sem1)
    sb.wait()
    c2 = pltpu.async_copy(xv.at[pl.ds(HLF, HLF)], ein_hbm.at[i0v.at[1]], sem2)
    c3 = pltpu.async_copy(xv.at[pl.ds(HLF, HLF)], ein_hbm.at[i1v.at[1]], sem3)
    c0.wait()
    c1.wait()
    c2.wait()
    c3.wait()


def _sc_dispatch(xt, d0, d1):
    mesh = plsc.VectorSubcoreMesh(core_axis_name="c", subcore_axis_name="s",
                                  num_cores=NC, num_subcores=NS)
    return pl.kernel(
        _sc_dispatch_body,
        out_type=jax.ShapeDtypeStruct((NROWS, D), jnp.float32),
        mesh=mesh,
        scratch_types=[
            pltpu.VMEM((TPW, D), jnp.float32),
            pltpu.VMEM((2, HLF), jnp.int32),
            pltpu.VMEM((2, HLF), jnp.int32),
            pltpu.SemaphoreType.DMA,
            pltpu.SemaphoreType.DMA,
            pltpu.SemaphoreType.DMA,
            pltpu.SemaphoreType.DMA,
            pltpu.SemaphoreType.DMA,
            pltpu.SemaphoreType.DMA,
        ],
    )(xt, d0, d1)


def _ffn_body(a_ref, w1_ref, b1_ref, w2_ref, b2_ref, y_ref):
    a = a_ref[...].astype(jnp.bfloat16)
    w1 = w1_ref[0].astype(jnp.bfloat16)
    h = jnp.maximum(
        lax.dot_general(a, w1, (((1,), (1,)), ((), ())),
                        preferred_element_type=jnp.float32)
        + b1_ref[0], 0.0).astype(jnp.bfloat16)
    w2 = w2_ref[0].astype(jnp.bfloat16)
    y_ref[...] = lax.dot_general(h, w2, (((1,), (1,)), ((), ())),
                                 preferred_element_type=jnp.float32) + b2_ref[0]


def _ffn(ein, w1, b1, w2, b2):
    return pl.pallas_call(
        _ffn_body,
        grid=(E,),
        in_specs=[
            pl.BlockSpec((EB, D), lambda e: (e, 0)),
            pl.BlockSpec((1, FF, D), lambda e: (e, 0, 0)),
            pl.BlockSpec((1, 1, FF), lambda e: (e, 0, 0)),
            pl.BlockSpec((1, D, FF), lambda e: (e, 0, 0)),
            pl.BlockSpec((1, 1, D), lambda e: (e, 0, 0)),
        ],
        out_specs=pl.BlockSpec((EB, D), lambda e: (e, 0)),
        out_shape=jax.ShapeDtypeStruct((NROWS, D), jnp.float32),
    )(ein, w1, b1.reshape(E, 1, FF), w2, b2.reshape(E, 1, D))


def _sc_combine_body(y_hbm, d0_hbm, d1_hbm, wr0_hbm, wr1_hbm, out_hbm,
                     r0v, r1v, i0v, i1v, w0v, w1v,
                     sem0, sem1, sem2, sem3, semo):
    wid = lax.axis_index("s") * NC + lax.axis_index("c")
    base = wid * TPW
    pltpu.sync_copy(d0_hbm.at[pl.ds(base, HLF)], i0v.at[0])
    pltpu.sync_copy(d0_hbm.at[pl.ds(base + HLF, HLF)], i0v.at[1])
    pltpu.sync_copy(d1_hbm.at[pl.ds(base, HLF)], i1v.at[0])
    pltpu.sync_copy(d1_hbm.at[pl.ds(base + HLF, HLF)], i1v.at[1])
    c0 = pltpu.async_copy(y_hbm.at[i0v.at[0]], r0v.at[pl.ds(0, HLF)], sem0)
    c1 = pltpu.async_copy(y_hbm.at[i1v.at[0]], r1v.at[pl.ds(0, HLF)], sem1)
    c2 = pltpu.async_copy(y_hbm.at[i0v.at[1]], r0v.at[pl.ds(HLF, HLF)], sem2)
    c3 = pltpu.async_copy(y_hbm.at[i1v.at[1]], r1v.at[pl.ds(HLF, HLF)], sem3)
    pltpu.sync_copy(wr0_hbm.at[pl.ds(base, TPW)], w0v)
    pltpu.sync_copy(wr1_hbm.at[pl.ds(base, TPW)], w1v)

    def body(j, carry):
        w0s = w0v[j, :]
        w1s = w1v[j, :]
        for c in range(D // 16):
            sl = pl.ds(c * 16, 16)
            r0v[j, sl] = w0s * r0v[j, sl] + w1s * r1v[j, sl]
        return carry

    c0.wait()
    c1.wait()
    lax.fori_loop(0, HLF, body, 0)
    oa = pltpu.async_copy(r0v.at[pl.ds(0, HLF)],
                          out_hbm.at[pl.ds(base, HLF)], semo)
    c2.wait()
    c3.wait()
    lax.fori_loop(HLF, TPW, body, 0)
    oa.wait()
    pltpu.sync_copy(r0v.at[pl.ds(HLF, HLF)],
                    out_hbm.at[pl.ds(base + HLF, HLF)])


def _sc_combine(y, d0, d1, wr0, wr1):
    mesh = plsc.VectorSubcoreMesh(core_axis_name="c", subcore_axis_name="s",
                                  num_cores=NC, num_subcores=NS)
    return pl.kernel(
        _sc_combine_body,
        out_type=jax.ShapeDtypeStruct((T, D), jnp.float32),
        mesh=mesh,
        scratch_types=[
            pltpu.VMEM((TPW, D), jnp.float32),
            pltpu.VMEM((TPW, D), jnp.float32),
            pltpu.VMEM((2, HLF), jnp.int32),
            pltpu.VMEM((2, HLF), jnp.int32),
            pltpu.VMEM((TPW, 16), jnp.float32),
            pltpu.VMEM((TPW, 16), jnp.float32),
            pltpu.SemaphoreType.DMA,
            pltpu.SemaphoreType.DMA,
            pltpu.SemaphoreType.DMA,
            pltpu.SemaphoreType.DMA,
            pltpu.SemaphoreType.DMA,
        ],
    )(y, d0, d1, wr0, wr1)


def kernel(x, Wg, bg, W1, B1, W2, B2):
    xt = x.reshape(T, D)
    d0c, d1c, wr0, wr1 = _route(xt, Wg, bg)
    d0 = d0c.reshape(T)
    d1 = d1c.reshape(T)
    ein = _sc_dispatch(xt, d0, d1)
    y = _ffn(ein, W1, B1, W2, B2)
    out = _sc_combine(y, d0, d1, wr0, wr1)
    return out.reshape(1, T, D)


# speedup vs baseline: 1.0207x; 1.0151x over previous
import functools
import math

import jax
import jax.numpy as jnp
from jax import lax
from jax.experimental import pallas as pl
from jax.experimental.pallas import tpu as pltpu
from jax.experimental.pallas import tpu_sc as plsc

E = 8
TOPK = 2
D = 768
FF = 3072
CF = 1.25
T = 2048
CAP = int(math.ceil(CF * T * TOPK / E))
SENT = E * CAP
NROWS = SENT + 8

NC, NS = 2, 16
NW = NC * NS
TPW = T // NW
RBLK = 512
FBLK = 3072
NFB = FF // FBLK


def _route_body(xt_ref, wg_ref, bg_ref, d0_ref, d1_ref, wr0_ref, wr1_ref,
                oh_s, meta_s, tblk_s):
    pid = pl.program_id(0)
    nblk = T // RBLK

    @pl.when(pid == 0)
    def _():
        xt = xt_ref[...]
        wg = wg_ref[...]
        logits = lax.dot_general(xt, wg, (((1,), (1,)), ((), ())),
                                 preferred_element_type=jnp.float32)
        logits = logits + bg_ref[...]
        iota8 = lax.broadcasted_iota(jnp.int32, (T, E), 1).astype(jnp.float32)
        m1 = jnp.max(logits, axis=1, keepdims=True)
        a1 = jnp.min(jnp.where(logits == m1, iota8, 1e9), axis=1, keepdims=True)
        masked = jnp.where(iota8 == a1, -1e30, logits)
        m2 = jnp.max(masked, axis=1, keepdims=True)
        a2 = jnp.min(jnp.where(masked == m2, iota8, 1e9), axis=1, keepdims=True)
        p = jnp.exp(logits - m1)
        denom = jnp.sum(p, axis=1, keepdims=True)
        g0 = 1.0 / denom
        g1 = jnp.exp(m2 - m1) / denom
        oh0 = (iota8 == a1).astype(jnp.float32)
        oh1 = (iota8 == a2).astype(jnp.float32)
        oh = jnp.concatenate([oh0, oh1], axis=1)
        oh_s[...] = oh
        meta_s[...] = jnp.concatenate([a1, a2, g0, g1], axis=1)
        srow = lax.broadcasted_iota(jnp.int32, (8, T), 0)
        scol = lax.broadcasted_iota(jnp.int32, (8, T), 1) // RBLK
        sel = (srow == scol).astype(jnp.float32)
        tblk_s[...] = lax.dot_general(sel, oh, (((1,), (0,)), ((), ())),
                                      preferred_element_type=jnp.float32)

    row = lax.broadcasted_iota(jnp.int32, (RBLK, RBLK), 0)
    col = lax.broadcasted_iota(jnp.int32, (RBLK, RBLK), 1)
    a_tri = (col <= row).astype(jnp.float32)
    ohb = oh_s[pl.ds(pid * RBLK, RBLK), :]
    c_loc = lax.dot_general(a_tri, ohb, (((1,), (0,)), ((), ())),
                            preferred_element_type=jnp.float32)
    tblk = tblk_s[...]
    bidx = lax.broadcasted_iota(jnp.int32, (8, 2 * E), 0)
    prefix = jnp.sum(jnp.where(bidx < pid, tblk, 0.0), axis=0, keepdims=True)
    totals = jnp.sum(tblk, axis=0, keepdims=True)
    c_inc = c_loc + prefix
    r0 = jnp.sum(c_inc[:, :E] * ohb[:, :E], axis=1, keepdims=True) - 1.0
    r1 = jnp.sum(c_inc[:, E:] * ohb[:, E:], axis=1, keepdims=True) - 1.0
    count0 = jnp.minimum(totals[:, :E], float(CAP))
    cnt_at = jnp.sum(ohb[:, E:] * count0, axis=1, keepdims=True)
    pos1 = r1 + cnt_at
    kept0 = r0 < CAP
    kept1 = pos1 < CAP
    meta = meta_s[pl.ds(pid * RBLK, RBLK), :]
    a1 = meta[:, 0:1]
    a2 = meta[:, 1:2]
    g0 = meta[:, 2:3]
    g1 = meta[:, 3:4]
    d0 = jnp.where(kept0, a1 * CAP + r0, float(SENT))
    d1 = jnp.where(kept1, a2 * CAP + pos1, float(SENT))
    d0_ref[...] = lax.transpose(d0.astype(jnp.int32), (1, 0))
    d1_ref[...] = lax.transpose(d1.astype(jnp.int32), (1, 0))
    wr0_ref[...] = jnp.broadcast_to(jnp.where(kept0, g0, 0.0), (RBLK, 16))
    wr1_ref[...] = jnp.broadcast_to(jnp.where(kept1, g1, 0.0), (RBLK, 16))


def _route(xt, wg, bg):
    d0, d1, wr0, wr1 = pl.pallas_call(
        _route_body,
        grid=(T // RBLK,),
        in_specs=[
            pl.BlockSpec((T, D), lambda i: (0, 0)),
            pl.BlockSpec((E, D), lambda i: (0, 0)),
            pl.BlockSpec((1, E), lambda i: (0, 0)),
        ],
        out_specs=(
            pl.BlockSpec((1, RBLK), lambda i: (0, i)),
            pl.BlockSpec((1, RBLK), lambda i: (0, i)),
            pl.BlockSpec((RBLK, 16), lambda i: (i, 0)),
            pl.BlockSpec((RBLK, 16), lambda i: (i, 0)),
        ),
        out_shape=(
            jax.ShapeDtypeStruct((1, T), jnp.int32),
            jax.ShapeDtypeStruct((1, T), jnp.int32),
            jax.ShapeDtypeStruct((T, 16), jnp.float32),
            jax.ShapeDtypeStruct((T, 16), jnp.float32),
        ),
        scratch_shapes=[
            pltpu.VMEM((T, 2 * E), jnp.float32),
            pltpu.VMEM((T, 4), jnp.float32),
            pltpu.VMEM((8, 2 * E), jnp.float32),
        ],
    )(xt, wg, bg.reshape(1, E))
    return d0, d1, wr0, wr1


HLF = TPW // 2


def _sc_dispatch_body(xt_hbm, d0_hbm, d1_hbm, ein_hbm, xv, i0v, i1v,
                      sema, semb, sem0, sem1, sem2, sem3):
    wid = lax.axis_index("s") * NC + lax.axis_index("c")
    base = wid * TPW
    sa = pltpu.async_copy(xt_hbm.at[pl.ds(base, HLF)], xv.at[pl.ds(0, HLF)],
                          sema)
    sb = pltpu.async_copy(xt_hbm.at[pl.ds(base + HLF, HLF)],
                          xv.at[pl.ds(HLF, HLF)], semb)
    pltpu.sync_copy(d0_hbm.at[pl.ds(base, HLF)], i0v.at[0])
    pltpu.sync_copy(d0_hbm.at[pl.ds(base + HLF, HLF)], i0v.at[1])
    pltpu.sync_copy(d1_hbm.at[pl.ds(base, HLF)], i1v.at[0])
    pltpu.sync_copy(d1_hbm.at[pl.ds(base + HLF, HLF)], i1v.at[1])
    sa.wait()
    c0 = pltpu.async_copy(xv.at[pl.ds(0, HLF)], ein_hbm.at[i0v.at[0]], sem0)
    c1 = pltpu.async_copy(xv.at[pl.ds(0, HLF)], ein_hbm.at[i1v.at[0]], sem1)
    sb.wait()
    c2 = pltpu.async_copy(xv.at[pl.ds(HLF, HLF)], ein_hbm.at[i0v.at[1]], sem2)
    c3 = pltpu.async_copy(xv.at[pl.ds(HLF, HLF)], ein_hbm.at[i1v.at[1]], sem3)
    c0.wait()
    c1.wait()
    c2.wait()
    c3.wait()


def _sc_dispatch(xt, d0, d1):
    mesh = plsc.VectorSubcoreMesh(core_axis_name="c", subcore_axis_name="s",
                                  num_cores=NC, num_subcores=NS)
    return pl.kernel(
        _sc_dispatch_body,
        out_type=jax.ShapeDtypeStruct((NROWS, D), jnp.float32),
        mesh=mesh,
        scratch_types=[
            pltpu.VMEM((TPW, D), jnp.float32),
            pltpu.VMEM((2, HLF), jnp.int32),
            pltpu.VMEM((2, HLF), jnp.int32),
            pltpu.SemaphoreType.DMA,
            pltpu.SemaphoreType.DMA,
            pltpu.SemaphoreType.DMA,
            pltpu.SemaphoreType.DMA,
            pltpu.SemaphoreType.DMA,
            pltpu.SemaphoreType.DMA,
        ],
    )(xt, d0, d1)


def _ffn_body(a_ref, w1_ref, b1_ref, w2_ref, b2_ref, y_ref):
    a = a_ref[...].astype(jnp.bfloat16)
    w1 = w1_ref[0].astype(jnp.bfloat16)
    h = jnp.maximum(
        lax.dot_general(a, w1, (((1,), (1,)), ((), ())),
                        preferred_element_type=jnp.float32)
        + b1_ref[0], 0.0).astype(jnp.bfloat16)
    w2 = w2_ref[0].astype(jnp.bfloat16)
    y_ref[...] = lax.dot_general(h, w2, (((1,), (1,)), ((), ())),
                                 preferred_element_type=jnp.float32) + b2_ref[0]


def _ffn(ein, w1, b1, w2, b2):
    return pl.pallas_call(
        _ffn_body,
        grid=(E,),
        in_specs=[
            pl.BlockSpec((CAP, D), lambda e: (e, 0)),
            pl.BlockSpec((1, FF, D), lambda e: (e, 0, 0)),
            pl.BlockSpec((1, 1, FF), lambda e: (e, 0, 0)),
            pl.BlockSpec((1, D, FF), lambda e: (e, 0, 0)),
            pl.BlockSpec((1, 1, D), lambda e: (e, 0, 0)),
        ],
        out_specs=pl.BlockSpec((CAP, D), lambda e: (e, 0)),
        out_shape=jax.ShapeDtypeStruct((NROWS, D), jnp.float32),
    )(ein, w1, b1.reshape(E, 1, FF), w2, b2.reshape(E, 1, D))


def _sc_combine_body(y_hbm, d0_hbm, d1_hbm, wr0_hbm, wr1_hbm, out_hbm,
                     r0v, r1v, i0v, i1v, w0v, w1v,
                     sem0, sem1, sem2, sem3, semo):
    wid = lax.axis_index("s") * NC + lax.axis_index("c")
    base = wid * TPW
    pltpu.sync_copy(d0_hbm.at[pl.ds(base, HLF)], i0v.at[0])
    pltpu.sync_copy(d0_hbm.at[pl.ds(base + HLF, HLF)], i0v.at[1])
    pltpu.sync_copy(d1_hbm.at[pl.ds(base, HLF)], i1v.at[0])
    pltpu.sync_copy(d1_hbm.at[pl.ds(base + HLF, HLF)], i1v.at[1])
    c0 = pltpu.async_copy(y_hbm.at[i0v.at[0]], r0v.at[pl.ds(0, HLF)], sem0)
    c1 = pltpu.async_copy(y_hbm.at[i1v.at[0]], r1v.at[pl.ds(0, HLF)], sem1)
    c2 = pltpu.async_copy(y_hbm.at[i0v.at[1]], r0v.at[pl.ds(HLF, HLF)], sem2)
    c3 = pltpu.async_copy(y_hbm.at[i1v.at[1]], r1v.at[pl.ds(HLF, HLF)], sem3)
    pltpu.sync_copy(wr0_hbm.at[pl.ds(base, TPW)], w0v)
    pltpu.sync_copy(wr1_hbm.at[pl.ds(base, TPW)], w1v)

    def body(j, carry):
        w0s = w0v[j, :]
        w1s = w1v[j, :]
        m0 = w0s != 0.0
        m1 = w1s != 0.0
        for c in range(D // 16):
            sl = pl.ds(c * 16, 16)
            v0 = r0v[j, sl]
            v1 = r1v[j, sl]
            r0v[j, sl] = (jnp.where(m0, w0s * v0, 0.0)
                          + jnp.where(m1, w1s * v1, 0.0))
        return carry

    c0.wait()
    c1.wait()
    lax.fori_loop(0, HLF, body, 0)
    oa = pltpu.async_copy(r0v.at[pl.ds(0, HLF)],
                          out_hbm.at[pl.ds(base, HLF)], semo)
    c2.wait()
    c3.wait()
    lax.fori_loop(HLF, TPW, body, 0)
    oa.wait()
    pltpu.sync_copy(r0v.at[pl.ds(HLF, HLF)],
                    out_hbm.at[pl.ds(base + HLF, HLF)])


def _sc_combine(y, d0, d1, wr0, wr1):
    mesh = plsc.VectorSubcoreMesh(core_axis_name="c", subcore_axis_name="s",
                                  num_cores=NC, num_subcores=NS)
    return pl.kernel(
        _sc_combine_body,
        out_type=jax.ShapeDtypeStruct((T, D), jnp.float32),
        mesh=mesh,
        scratch_types=[
            pltpu.VMEM((TPW, D), jnp.float32),
            pltpu.VMEM((TPW, D), jnp.float32),
            pltpu.VMEM((2, HLF), jnp.int32),
            pltpu.VMEM((2, HLF), jnp.int32),
            pltpu.VMEM((TPW, 16), jnp.float32),
            pltpu.VMEM((TPW, 16), jnp.float32),
            pltpu.SemaphoreType.DMA,
            pltpu.SemaphoreType.DMA,
            pltpu.SemaphoreType.DMA,
            pltpu.SemaphoreType.DMA,
            pltpu.SemaphoreType.DMA,
        ],
    )(y, d0, d1, wr0, wr1)


def kernel(x, Wg, bg, W1, B1, W2, B2):
    xt = x.reshape(T, D)
    d0c, d1c, wr0, wr1 = _route(xt, Wg, bg)
    d0 = d0c.reshape(T)
    d1 = d1c.reshape(T)
    ein = _sc_dispatch(xt, d0, d1)
    y = _ffn(ein, W1, B1, W2, B2)
    out = _sc_combine(y, d0, d1, wr0, wr1)
    return out.reshape(1, T, D)
